# baseline (speedup 1.0000x reference)
import math

import jax
import jax.numpy as jnp
import numpy as np
from jax.experimental import pallas as pl

_NBINS = 9
_H = 224
_POOL = 8
_HP = _H // _POOL


def _upsample_matrix() -> np.ndarray:
    u = np.zeros((_H, _HP), dtype=np.float32)
    for y in range(_H):
        fy = (y + 0.5) / _POOL - 0.5
        y0 = math.floor(fy)
        w = fy - y0
        u[y, min(max(y0, 0), _HP - 1)] += 1.0 - w
        u[y, min(max(y0 + 1, 0), _HP - 1)] += w
    return u


def _pool_matrix() -> np.ndarray:
    p = np.zeros((_HP, _H), dtype=np.float32)
    for i in range(_H):
        p[i // _POOL, i] = 1.0
    return p


def _block_diag(m: np.ndarray, n: int) -> np.ndarray:
    r, c = m.shape
    out = np.zeros((n * r, n * c), dtype=np.float32)
    for i in range(n):
        out[i * r:(i + 1) * r, i * c:(i + 1) * c] = m
    return out


_U = _upsample_matrix()
_P = _pool_matrix()
_BDP = _block_diag(_P, _NBINS)
_BDU = _block_diag(_U, _NBINS)

_COS = [math.cos(k * math.pi / _NBINS) for k in range(1, _NBINS)]
_SIN = [math.sin(k * math.pi / _NBINS) for k in range(1, _NBINS)]

_HIGH = jax.lax.Precision.HIGHEST


def _hog_body(x_ref, tg_ref, pt_ref, bdp_ref, bdu_ref, ut_ref, o_ref):
    img = x_ref[0, 0].astype(jnp.bfloat16).astype(jnp.float32)

    xp = jnp.concatenate([img[1:2], img, img[_H - 2:_H - 1]], axis=0)
    xp = jnp.concatenate([xp[:, 1:2], xp, xp[:, _H - 2:_H - 1]], axis=1)

    colsum = xp[0:_H] + 2.0 * xp[1:_H + 1] + xp[2:_H + 2]
    gx = colsum[:, 0:_H] - colsum[:, 2:_H + 2]
    rowdiff = xp[0:_H] - xp[2:_H + 2]
    gy = rowdiff[:, 0:_H] + 2.0 * rowdiff[:, 1:_H + 1] + rowdiff[:, 2:_H + 2]

    mag = jnp.sqrt(gx * gx + gy * gy) * tg_ref[...]

    neg = jnp.logical_or(gx < 0.0,
                         jnp.logical_and(gx == 0.0, gy < 0.0))
    ts = [jnp.logical_xor(gx * c - gy * s >= 0.0, neg)
          for c, s in zip(_COS, _SIN)]
    masks = [jnp.logical_not(ts[0])]
    masks += [jnp.logical_and(ts[k], jnp.logical_not(ts[k + 1]))
              for k in range(_NBINS - 2)]
    masks.append(ts[_NBINS - 2])

    zero = jnp.zeros_like(mag)
    sel = jnp.concatenate([jnp.where(m, mag, zero) for m in masks], axis=0)

    a = jnp.dot(sel, pt_ref[...], precision=_HIGH,
                preferred_element_type=jnp.float32)
    h = jnp.dot(bdp_ref[...], a, precision=_HIGH,
                preferred_element_type=jnp.float32)
    r = jnp.dot(bdu_ref[...], h, precision=_HIGH,
                preferred_element_type=jnp.float32)
    out = jnp.dot(r, ut_ref[...], precision=_HIGH,
                  preferred_element_type=jnp.float32)
    o_ref[0] = out.reshape(_NBINS, _H, _H)


def kernel(x, weight_x, weight_y, gkern):
    b = x.shape[0]
    tg = jnp.tile(gkern, (_H // gkern.shape[0], _H // gkern.shape[1]))

    out = pl.pallas_call(
        _hog_body,
        grid=(b,),
        in_specs=[
            pl.BlockSpec((1, 1, _H, _H), lambda i: (i, 0, 0, 0)),
            pl.BlockSpec((_H, _H), lambda i: (0, 0)),
            pl.BlockSpec((_H, _HP), lambda i: (0, 0)),
            pl.BlockSpec(_BDP.shape, lambda i: (0, 0)),
            pl.BlockSpec(_BDU.shape, lambda i: (0, 0)),
            pl.BlockSpec((_HP, _H), lambda i: (0, 0)),
        ],
        out_specs=pl.BlockSpec((1, _NBINS, _H, _H), lambda i: (i, 0, 0, 0)),
        out_shape=jax.ShapeDtypeStruct((b, _NBINS, _H, _H), jnp.float32),
    )(x, tg, jnp.asarray(_P.T), jnp.asarray(_BDP), jnp.asarray(_BDU),
      jnp.asarray(_U.T))
    return out


# speedup vs baseline: 4.6649x; 4.6649x over previous
import math

import jax
import jax.numpy as jnp
import numpy as np
from jax.experimental import pallas as pl

_NBINS = 9
_H = 224
_POOL = 8
_HP = _H // _POOL


def _upsample_matrix() -> np.ndarray:
    u = np.zeros((_H, _HP), dtype=np.float32)
    for y in range(_H):
        fy = (y + 0.5) / _POOL - 0.5
        y0 = math.floor(fy)
        w = fy - y0
        u[y, min(max(y0, 0), _HP - 1)] += 1.0 - w
        u[y, min(max(y0 + 1, 0), _HP - 1)] += w
    return u


def _pool_matrix() -> np.ndarray:
    p = np.zeros((_HP, _H), dtype=np.float32)
    for i in range(_H):
        p[i // _POOL, i] = 1.0
    return p


def _block_diag(m: np.ndarray, n: int) -> np.ndarray:
    r, c = m.shape
    out = np.zeros((n * r, n * c), dtype=np.float32)
    for i in range(n):
        out[i * r:(i + 1) * r, i * c:(i + 1) * c] = m
    return out


_U = _upsample_matrix()
_P = _pool_matrix()
_BDP = _block_diag(_P, _NBINS)
_BDU = _block_diag(_U, _NBINS)

_COS = [math.cos(k * math.pi / _NBINS) for k in range(1, _NBINS)]
_SIN = [math.sin(k * math.pi / _NBINS) for k in range(1, _NBINS)]

_HIGH = jax.lax.Precision.HIGHEST


def _hog_body(x_ref, tg_ref, pt_ref, bdp_ref, bdu_ref, ut_ref, o_ref):
    img = x_ref[0, 0].astype(jnp.bfloat16).astype(jnp.float32)

    xp = jnp.concatenate([img[1:2], img, img[_H - 2:_H - 1]], axis=0)
    xp = jnp.concatenate([xp[:, 1:2], xp, xp[:, _H - 2:_H - 1]], axis=1)

    colsum = xp[0:_H] + 2.0 * xp[1:_H + 1] + xp[2:_H + 2]
    gx = colsum[:, 0:_H] - colsum[:, 2:_H + 2]
    rowdiff = xp[0:_H] - xp[2:_H + 2]
    gy = rowdiff[:, 0:_H] + 2.0 * rowdiff[:, 1:_H + 1] + rowdiff[:, 2:_H + 2]

    mag = jnp.sqrt(gx * gx + gy * gy) * tg_ref[...]

    neg = jnp.logical_or(gx < 0.0,
                         jnp.logical_and(gx == 0.0, gy < 0.0))
    ts = [jnp.logical_xor(gx * c - gy * s >= 0.0, neg)
          for c, s in zip(_COS, _SIN)]
    masks = [jnp.logical_not(ts[0])]
    masks += [jnp.logical_and(ts[k], jnp.logical_not(ts[k + 1]))
              for k in range(_NBINS - 2)]
    masks.append(ts[_NBINS - 2])

    zero = jnp.zeros_like(mag)
    sel = jnp.concatenate([jnp.where(m, mag, zero) for m in masks], axis=0)

    a = jnp.dot(sel, pt_ref[...],
                preferred_element_type=jnp.float32)
    h = jnp.dot(bdp_ref[...], a,
                preferred_element_type=jnp.float32)
    r = jnp.dot(bdu_ref[...], h,
                preferred_element_type=jnp.float32)
    out = jnp.dot(r, ut_ref[...],
                  preferred_element_type=jnp.float32)
    o_ref[0] = out.reshape(_NBINS, _H, _H)


def kernel(x, weight_x, weight_y, gkern):
    b = x.shape[0]
    tg = jnp.tile(gkern, (_H // gkern.shape[0], _H // gkern.shape[1]))

    out = pl.pallas_call(
        _hog_body,
        grid=(b,),
        in_specs=[
            pl.BlockSpec((1, 1, _H, _H), lambda i: (i, 0, 0, 0)),
            pl.BlockSpec((_H, _H), lambda i: (0, 0)),
            pl.BlockSpec((_H, _HP), lambda i: (0, 0)),
            pl.BlockSpec(_BDP.shape, lambda i: (0, 0)),
            pl.BlockSpec(_BDU.shape, lambda i: (0, 0)),
            pl.BlockSpec((_HP, _H), lambda i: (0, 0)),
        ],
        out_specs=pl.BlockSpec((1, _NBINS, _H, _H), lambda i: (i, 0, 0, 0)),
        out_shape=jax.ShapeDtypeStruct((b, _NBINS, _H, _H), jnp.float32),
    )(x, tg, jnp.asarray(_P.T), jnp.asarray(_BDP), jnp.asarray(_BDU),
      jnp.asarray(_U.T))
    return out


# speedup vs baseline: 7.0016x; 1.5009x over previous
import math

import jax
import jax.numpy as jnp
import numpy as np
from jax.experimental import pallas as pl

_NBINS = 9
_H = 224
_POOL = 8
_HP = _H // _POOL


def _upsample_matrix() -> np.ndarray:
    u = np.zeros((_H, _HP), dtype=np.float32)
    for y in range(_H):
        fy = (y + 0.5) / _POOL - 0.5
        y0 = math.floor(fy)
        w = fy - y0
        u[y, min(max(y0, 0), _HP - 1)] += 1.0 - w
        u[y, min(max(y0 + 1, 0), _HP - 1)] += w
    return u


def _pool_matrix() -> np.ndarray:
    p = np.zeros((_HP, _H), dtype=np.float32)
    for i in range(_H):
        p[i // _POOL, i] = 1.0
    return p


def _block_diag(m: np.ndarray, n: int) -> np.ndarray:
    r, c = m.shape
    out = np.zeros((n * r, n * c), dtype=np.float32)
    for i in range(n):
        out[i * r:(i + 1) * r, i * c:(i + 1) * c] = m
    return out


_U = _upsample_matrix()
_P = _pool_matrix()
_BDP = _block_diag(_P, _NBINS)
_BDU = _block_diag(_U, _NBINS)

_COS = [math.cos(k * math.pi / _NBINS) for k in range(1, _NBINS)]
_SIN = [math.sin(k * math.pi / _NBINS) for k in range(1, _NBINS)]

_HIGH = jax.lax.Precision.HIGHEST


def _hog_body(x_ref, tg_ref, pt_ref, bdu_ref, ut_ref, o_ref):
    img = x_ref[0, 0].astype(jnp.bfloat16).astype(jnp.float32)

    xp = jnp.concatenate([img[1:2], img, img[_H - 2:_H - 1]], axis=0)
    xp = jnp.concatenate([xp[:, 1:2], xp, xp[:, _H - 2:_H - 1]], axis=1)

    colsum = xp[0:_H] + 2.0 * xp[1:_H + 1] + xp[2:_H + 2]
    gx = colsum[:, 0:_H] - colsum[:, 2:_H + 2]
    rowdiff = xp[0:_H] - xp[2:_H + 2]
    gy = rowdiff[:, 0:_H] + 2.0 * rowdiff[:, 1:_H + 1] + rowdiff[:, 2:_H + 2]

    mag = jnp.sqrt(gx * gx + gy * gy) * tg_ref[...]

    neg = jnp.logical_or(gx < 0.0,
                         jnp.logical_and(gx == 0.0, gy < 0.0))
    flip = jnp.where(neg, -1.0, 1.0)
    gxf = gx * flip
    gyf = gy * flip
    one = jnp.ones_like(mag)
    zero = jnp.zeros_like(mag)
    fs = [one]
    fs += [jnp.where(gxf * c - gyf * s >= 0.0, one, zero)
           for c, s in zip(_COS, _SIN)]
    fs.append(zero)
    sel = jnp.concatenate([mag * (fs[k] - fs[k + 1])
                           for k in range(_NBINS)], axis=0)

    a = jnp.dot(sel, pt_ref[...],
                preferred_element_type=jnp.float32)
    a3 = a.reshape(252, 8, _HP)
    h = a3.sum(axis=1)
    r = jnp.dot(bdu_ref[...], h,
                preferred_element_type=jnp.float32)
    out = jnp.dot(r, ut_ref[...],
                  preferred_element_type=jnp.float32)
    o_ref[0] = out.reshape(_NBINS, _H, _H)


def kernel(x, weight_x, weight_y, gkern):
    b = x.shape[0]
    tg = jnp.tile(gkern, (_H // gkern.shape[0], _H // gkern.shape[1]))

    out = pl.pallas_call(
        _hog_body,
        grid=(b,),
        in_specs=[
            pl.BlockSpec((1, 1, _H, _H), lambda i: (i, 0, 0, 0)),
            pl.BlockSpec((_H, _H), lambda i: (0, 0)),
            pl.BlockSpec((_H, _HP), lambda i: (0, 0)),
            pl.BlockSpec(_BDU.shape, lambda i: (0, 0)),
            pl.BlockSpec((_HP, _H), lambda i: (0, 0)),
        ],
        out_specs=pl.BlockSpec((1, _NBINS, _H, _H), lambda i: (i, 0, 0, 0)),
        out_shape=jax.ShapeDtypeStruct((b, _NBINS, _H, _H), jnp.float32),
    )(x, tg, jnp.asarray(_P.T), jnp.asarray(_BDU), jnp.asarray(_U.T))
    return out


# speedup vs baseline: 7.5073x; 1.0722x over previous
import math

import jax
import jax.numpy as jnp
import numpy as np
from jax.experimental import pallas as pl

_NBINS = 9
_H = 224
_POOL = 8
_HP = _H // _POOL


def _upsample_matrix() -> np.ndarray:
    u = np.zeros((_H, _HP), dtype=np.float32)
    for y in range(_H):
        fy = (y + 0.5) / _POOL - 0.5
        y0 = math.floor(fy)
        w = fy - y0
        u[y, min(max(y0, 0), _HP - 1)] += 1.0 - w
        u[y, min(max(y0 + 1, 0), _HP - 1)] += w
    return u


def _pool_matrix() -> np.ndarray:
    p = np.zeros((_HP, _H), dtype=np.float32)
    for i in range(_H):
        p[i // _POOL, i] = 1.0
    return p


def _block_diag(m: np.ndarray, n: int) -> np.ndarray:
    r, c = m.shape
    out = np.zeros((n * r, n * c), dtype=np.float32)
    for i in range(n):
        out[i * r:(i + 1) * r, i * c:(i + 1) * c] = m
    return out


_U = _upsample_matrix()
_P = _pool_matrix()
_BDP = _block_diag(_P, _NBINS)
_BDU = _block_diag(_U, _NBINS)

_COS = [math.cos(k * math.pi / _NBINS) for k in range(1, _NBINS)]
_SIN = [math.sin(k * math.pi / _NBINS) for k in range(1, _NBINS)]

_HIGH = jax.lax.Precision.HIGHEST


def _hog_body(x_ref, tg_ref, pt_ref, bdu_ref, ut_ref, o_ref):
    img = x_ref[0, 0].astype(jnp.bfloat16).astype(jnp.float32)

    xp = jnp.concatenate([img[1:2], img, img[_H - 2:_H - 1]], axis=0)
    xp = jnp.concatenate([xp[:, 1:2], xp, xp[:, _H - 2:_H - 1]], axis=1)

    colsum = xp[0:_H] + 2.0 * xp[1:_H + 1] + xp[2:_H + 2]
    gx = colsum[:, 0:_H] - colsum[:, 2:_H + 2]
    rowdiff = xp[0:_H] - xp[2:_H + 2]
    gy = rowdiff[:, 0:_H] + 2.0 * rowdiff[:, 1:_H + 1] + rowdiff[:, 2:_H + 2]

    mag = jnp.sqrt(gx * gx + gy * gy) * tg_ref[...]

    neg = jnp.logical_or(gx < 0.0,
                         jnp.logical_and(gx == 0.0, gy < 0.0))
    flip = jnp.where(neg, -1.0, 1.0)
    gxf = gx * flip
    gyf = gy * flip
    ts = [None] * (_NBINS + 1)
    for k in range(1, 5):
        p = gxf * _COS[k - 1]
        q = gyf * _SIN[k - 1]
        ts[k] = (p - q) >= 0.0
        ts[_NBINS - k] = (p + q) <= 0.0
    zero = jnp.zeros_like(mag)
    sel = jnp.concatenate([mag] + [jnp.where(ts[k], mag, zero)
                                   for k in range(1, _NBINS)], axis=0)

    b = jnp.dot(sel, pt_ref[...],
                preferred_element_type=jnp.float32)
    a = b - jnp.concatenate([b[_H:], jnp.zeros((_H, _HP), jnp.float32)],
                            axis=0)
    a3 = a.reshape(252, 8, _HP)
    h = a3.sum(axis=1)
    r = jnp.dot(bdu_ref[...], h,
                preferred_element_type=jnp.float32)
    out = jnp.dot(r, ut_ref[...],
                  preferred_element_type=jnp.float32)
    o_ref[0] = out.reshape(_NBINS, _H, _H)


def kernel(x, weight_x, weight_y, gkern):
    b = x.shape[0]
    tg = jnp.tile(gkern, (_H // gkern.shape[0], _H // gkern.shape[1]))

    out = pl.pallas_call(
        _hog_body,
        grid=(b,),
        in_specs=[
            pl.BlockSpec((1, 1, _H, _H), lambda i: (i, 0, 0, 0)),
            pl.BlockSpec((_H, _H), lambda i: (0, 0)),
            pl.BlockSpec((_H, _HP), lambda i: (0, 0)),
            pl.BlockSpec(_BDU.shape, lambda i: (0, 0)),
            pl.BlockSpec((_HP, _H), lambda i: (0, 0)),
        ],
        out_specs=pl.BlockSpec((1, _NBINS, _H, _H), lambda i: (i, 0, 0, 0)),
        out_shape=jax.ShapeDtypeStruct((b, _NBINS, _H, _H), jnp.float32),
    )(x, tg, jnp.asarray(_P.T), jnp.asarray(_BDU), jnp.asarray(_U.T))
    return out
